# baseline (speedup 1.0000x reference)
import functools

import jax
import jax.numpy as jnp
from jax import lax
from jax.experimental import pallas as pl
from jax.experimental.pallas import tpu as pltpu
from jax.experimental.pallas import tpu_sc as plsc

_LANES = 16


@functools.lru_cache(maxsize=None)
def _make_jitter_kernel(n_batch, n_chan, n_in, n_out):
    NC = 2
    NS = 16
    NW = NC * NS
    c_split = NW // n_batch
    assert n_batch * c_split == NW
    CW = n_chan // c_split
    assert CW * c_split == n_chan
    TW = 128
    n_chunks = n_out // TW
    assert TW * n_chunks == n_out
    assert n_in == n_out + 2
    n_vec = TW // _LANES

    mesh = plsc.VectorSubcoreMesh(core_axis_name="c", subcore_axis_name="s")

    @functools.partial(
        pl.kernel,
        out_type=jax.ShapeDtypeStruct((n_batch * n_chan, n_out), jnp.float32),
        mesh=mesh,
        compiler_params=pltpu.CompilerParams(needs_layout_passes=False),
        scratch_types=[
            pltpu.VMEM((n_in,), jnp.int32),
            pltpu.VMEM((TW + 2, CW), jnp.float32),
            pltpu.VMEM((TW + 2, CW), jnp.float32),
            pltpu.VMEM((CW, TW), jnp.float32),
            pltpu.SemaphoreType.DMA,
            pltpu.SemaphoreType.DMA,
            pltpu.SemaphoreType.DMA,
        ],
    )
    def jitter(xt_hbm, idx_hbm, out_hbm, idx_v, xbuf0, xbuf1, obuf,
               sem_i0, sem_i1, sem_o):
        wid = lax.axis_index("s") * NC + lax.axis_index("c")
        b = wid // c_split
        c0 = (wid % c_split) * CW
        row0 = b * n_chan + c0
        pltpu.sync_copy(idx_hbm.at[b], idx_v)

        xbufs = (xbuf0, xbuf1)
        sems_i = (sem_i0, sem_i1)

        def start_in(ch):
            return pltpu.async_copy(
                xt_hbm.at[pl.ds(ch * TW, TW + 2), b, pl.ds(c0, CW)],
                xbufs[ch % 2], sems_i[ch % 2])

        def start_out(ch):
            return pltpu.async_copy(
                obuf, out_hbm.at[pl.ds(row0, CW), pl.ds(ch * TW, TW)], sem_o)

        in_h = {0: start_in(0)}
        out_h = {}
        for ch in range(n_chunks):
            if ch + 1 < n_chunks:
                in_h[ch + 1] = start_in(ch + 1)
            in_h.pop(ch).wait()
            if ch >= 1:
                out_h.pop(ch - 1).wait()
            xbuf = xbufs[ch % 2]
            t0 = ch * TW
            ivs = [idx_v[pl.ds(t0 + v * _LANES + 1, _LANES)] - t0
                   for v in range(n_vec)]

            @plsc.parallel_loop(0, CW, unroll=4)
            def gather_body(k):
                ksplat = jnp.full((_LANES,), k, jnp.int32)
                for v in range(n_vec):
                    obuf[k, pl.ds(v * _LANES, _LANES)] = plsc.load_gather(
                        xbuf, [ivs[v], ksplat])

            out_h[ch] = start_out(ch)
        for ch in sorted(out_h):
            out_h.pop(ch).wait()

    return jitter


def kernel(x, mindex):
    B, C, T2 = x.shape
    T = T2 - 2
    idx = mindex if mindex.dtype == jnp.int32 else mindex.astype(jnp.int32)
    xt = jnp.transpose(x, (2, 0, 1))
    out = _make_jitter_kernel(B, C, T2, T)(xt, idx)
    return out.reshape(B, C, T)
